# baseline (speedup 1.0000x reference)
import jax
import jax.numpy as jnp
from jax.experimental import pallas as pl

_GRID_SIZE = 4
_ORDER = 3
_H = 0.5
_KNOTS = [_H * i - 2.5 for i in range(_GRID_SIZE + 2 * _ORDER + 1)]


def _spline_bases(y):
    b = [jnp.maximum(1.0 - 2.0 * jnp.abs(y - _KNOTS[i + 1]), 0.0)
         for i in range(len(_KNOTS) - 2)]
    for j in range(2, _ORDER + 1):
        inv = 1.0 / (j * _H)
        z = y * inv
        u = [z - _KNOTS[i] * inv for i in range(len(b))]
        b = [u[i] * b[i] + (1.0 - u[i + 1]) * b[i + 1]
             for i in range(len(b) - 1)]
    return b


def _silu_and_bases(y):
    s16 = jax.nn.silu(y).astype(jnp.bfloat16)
    bs16 = jnp.concatenate(_spline_bases(y), axis=1).astype(jnp.bfloat16)
    return s16, bs16


def _dot(a, b):
    return jnp.dot(a, b, preferred_element_type=jnp.float32)


def _pass1_kernel(a_ref, f_ref, bw_ref, sw_ref, bwp_ref, swp_ref,
                  h16_ref, part_ref):
    y = _dot(a_ref[...].astype(jnp.bfloat16), f_ref[...])
    s16, bs16 = _silu_and_bases(y)
    h = jnp.maximum(_dot(s16, bw_ref[...]) + _dot(bs16, sw_ref[...]), 0.0)
    h16_ref[...] = h.astype(jnp.bfloat16)
    part_ref[...] = _dot(s16, bwp_ref[...]) + _dot(bs16, swp_ref[...])


def _pass2_kernel(a_ref, f_ref, bw_ref, sw_ref, bwp_ref, swp_ref, pin_ref,
                  h16_ref, part_ref):
    y = _dot(a_ref[...].astype(jnp.bfloat16), f_ref[...])
    s16, bs16 = _silu_and_bases(y)
    h = jnp.maximum(_dot(s16, bw_ref[...]) + _dot(bs16, sw_ref[...]), 0.0)
    h16_ref[...] = h.astype(jnp.bfloat16)
    part_ref[...] = (pin_ref[...] + _dot(s16, bwp_ref[...])
                     + _dot(bs16, swp_ref[...]))


def _pass3_kernel(a_ref, f_ref, bwp_ref, swp_ref, pin_ref, o_ref):
    y3 = _dot(a_ref[...].astype(jnp.bfloat16), f_ref[...])
    s16, bs16 = _silu_and_bases(y3)
    o_ref[...] = jnp.maximum(
        pin_ref[...] + _dot(s16, bwp_ref[...]) + _dot(bs16, swp_ref[...]), 0.0)


def _prep_spline_w(spline_w, scaler):
    sw = spline_w * scaler[:, :, None]
    w = sw.transpose(2, 1, 0).reshape(-1, sw.shape[0])
    return w.astype(jnp.bfloat16)


def _full(shape):
    return pl.BlockSpec(shape, lambda i: (0, 0))


def kernel(x, edge_index, base_w1, spline_w1, scaler1, base_w2, spline_w2,
           scaler2, base_wo, spline_wo, scaler_o):
    n, f = x.shape
    h_dim = base_w1.shape[0]
    c_dim = base_wo.shape[0]
    bm = 400
    assert n % bm == 0
    grid = (n // bm,)

    x16 = x.astype(jnp.bfloat16)
    bw1 = base_w1.T.astype(jnp.bfloat16)
    bw2 = base_w2.T.astype(jnp.bfloat16)
    sw1 = _prep_spline_w(spline_w1, scaler1)
    sw2 = _prep_spline_w(spline_w2, scaler2)
    bwp = [base_wo[:, k * f:(k + 1) * f].T.astype(jnp.bfloat16)
           for k in range(3)]
    swp = [_prep_spline_w(spline_wo[:, k * f:(k + 1) * f, :],
                          scaler_o[:, k * f:(k + 1) * f]) for k in range(3)]

    row_blk = pl.BlockSpec((bm, n), lambda i: (i, 0))
    h_blk = pl.BlockSpec((bm, h_dim), lambda i: (i, 0))
    p_blk = pl.BlockSpec((bm, c_dim), lambda i: (i, 0))

    h16, p1 = pl.pallas_call(
        _pass1_kernel,
        grid=grid,
        in_specs=[row_blk, _full((n, f)), _full(bw1.shape), _full(sw1.shape),
                  _full(bwp[0].shape), _full(swp[0].shape)],
        out_specs=[h_blk, p_blk],
        out_shape=[jax.ShapeDtypeStruct((n, h_dim), jnp.bfloat16),
                   jax.ShapeDtypeStruct((n, c_dim), jnp.float32)],
    )(edge_index, x16, bw1, sw1, bwp[0], swp[0])

    h2_16, p12 = pl.pallas_call(
        _pass2_kernel,
        grid=grid,
        in_specs=[row_blk, _full((n, h_dim)), _full(bw2.shape),
                  _full(sw2.shape), _full(bwp[1].shape), _full(swp[1].shape),
                  p_blk],
        out_specs=[h_blk, p_blk],
        out_shape=[jax.ShapeDtypeStruct((n, h_dim), jnp.bfloat16),
                   jax.ShapeDtypeStruct((n, c_dim), jnp.float32)],
    )(edge_index, h16, bw2, sw2, bwp[1], swp[1], p1)

    out = pl.pallas_call(
        _pass3_kernel,
        grid=grid,
        in_specs=[row_blk, _full((n, h_dim)), _full(bwp[2].shape),
                  _full(swp[2].shape), p_blk],
        out_specs=p_blk,
        out_shape=jax.ShapeDtypeStruct((n, c_dim), jnp.float32),
    )(edge_index, h2_16, bwp[2], swp[2], p12)
    return out


# speedup vs baseline: 1.0046x; 1.0046x over previous
import jax
import jax.numpy as jnp
from jax.experimental import pallas as pl
from jax.experimental.pallas import tpu as pltpu

_GRID_SIZE = 4
_ORDER = 3
_H = 0.5
_KNOTS = [_H * i - 2.5 for i in range(_GRID_SIZE + 2 * _ORDER + 1)]


def _spline_bases(y):
    b = [jnp.maximum(1.0 - 2.0 * jnp.abs(y - _KNOTS[i + 1]), 0.0)
         for i in range(len(_KNOTS) - 2)]
    for j in range(2, _ORDER + 1):
        inv = 1.0 / (j * _H)
        z = y * inv
        u = [z - _KNOTS[i] * inv for i in range(len(b))]
        b = [u[i] * b[i] + (1.0 - u[i + 1]) * b[i + 1]
             for i in range(len(b) - 1)]
    return b


def _silu_and_bases(y):
    s16 = jax.nn.silu(y).astype(jnp.bfloat16)
    bs16 = jnp.concatenate(_spline_bases(y), axis=1).astype(jnp.bfloat16)
    return s16, bs16


def _dot(a, b):
    return jnp.dot(a, b, preferred_element_type=jnp.float32)


def _make_fused_kernel(nblk, bm):
    def fused(a_ref, x16_ref, bw1_ref, sw1_ref, bw2_ref, sw2_ref,
              bwp0_ref, swp0_ref, bwp1_ref, swp1_ref, bwp2_ref, swp2_ref,
              o_ref, h16_scr, h2_scr, part_scr):
        i = pl.program_id(0)
        phase = i // nblk
        rows = pl.ds((i % nblk) * bm, bm)
        a16 = a_ref[...].astype(jnp.bfloat16)

        @pl.when(phase == 0)
        def _():
            y = _dot(a16, x16_ref[...])
            s16, bs16 = _silu_and_bases(y)
            h = jnp.maximum(_dot(s16, bw1_ref[...]) + _dot(bs16, sw1_ref[...]),
                            0.0)
            h16_scr[rows, :] = h.astype(jnp.bfloat16)
            part_scr[rows, :] = (_dot(s16, bwp0_ref[...])
                                 + _dot(bs16, swp0_ref[...]))

        @pl.when(phase == 1)
        def _():
            y = _dot(a16, h16_scr[...])
            s16, bs16 = _silu_and_bases(y)
            h2 = jnp.maximum(_dot(s16, bw2_ref[...]) + _dot(bs16, sw2_ref[...]),
                             0.0)
            h2_scr[rows, :] = h2.astype(jnp.bfloat16)
            part_scr[rows, :] += (_dot(s16, bwp1_ref[...])
                                  + _dot(bs16, swp1_ref[...]))

        @pl.when(phase == 2)
        def _():
            y3 = _dot(a16, h2_scr[...])
            s16, bs16 = _silu_and_bases(y3)
            o_ref[...] = jnp.maximum(
                part_scr[rows, :] + _dot(s16, bwp2_ref[...])
                + _dot(bs16, swp2_ref[...]), 0.0)

    return fused


def _prep_spline_w(spline_w, scaler):
    sw = spline_w * scaler[:, :, None]
    w = sw.transpose(2, 1, 0).reshape(-1, sw.shape[0])
    return w.astype(jnp.bfloat16)


def _full(shape):
    return pl.BlockSpec(shape, lambda i: (0, 0))


def kernel(x, edge_index, base_w1, spline_w1, scaler1, base_w2, spline_w2,
           scaler2, base_wo, spline_wo, scaler_o):
    n, f = x.shape
    h_dim = base_w1.shape[0]
    c_dim = base_wo.shape[0]
    bm = 400
    assert n % bm == 0
    nblk = n // bm

    x16 = x.astype(jnp.bfloat16)
    bw1 = base_w1.T.astype(jnp.bfloat16)
    bw2 = base_w2.T.astype(jnp.bfloat16)
    sw1 = _prep_spline_w(spline_w1, scaler1)
    sw2 = _prep_spline_w(spline_w2, scaler2)
    bwp = [base_wo[:, k * f:(k + 1) * f].T.astype(jnp.bfloat16)
           for k in range(3)]
    swp = [_prep_spline_w(spline_wo[:, k * f:(k + 1) * f, :],
                          scaler_o[:, k * f:(k + 1) * f]) for k in range(3)]

    row_blk = pl.BlockSpec((bm, n), lambda i: (i % nblk, 0))

    out = pl.pallas_call(
        _make_fused_kernel(nblk, bm),
        grid=(3 * nblk,),
        in_specs=[row_blk, _full((n, f)), _full(bw1.shape), _full(sw1.shape),
                  _full(bw2.shape), _full(sw2.shape),
                  _full(bwp[0].shape), _full(swp[0].shape),
                  _full(bwp[1].shape), _full(swp[1].shape),
                  _full(bwp[2].shape), _full(swp[2].shape)],
        out_specs=pl.BlockSpec((bm, c_dim), lambda i: (i % nblk, 0)),
        out_shape=jax.ShapeDtypeStruct((n, c_dim), jnp.float32),
        scratch_shapes=[pltpu.VMEM((n, h_dim), jnp.bfloat16),
                        pltpu.VMEM((n, h_dim), jnp.bfloat16),
                        pltpu.VMEM((n, c_dim), jnp.float32)],
    )(edge_index, x16, bw1, sw1, bw2, sw2,
      bwp[0], swp[0], bwp[1], swp[1], bwp[2], swp[2])
    return out


# speedup vs baseline: 1.0787x; 1.0737x over previous
import jax
import jax.numpy as jnp
from jax.experimental import pallas as pl
from jax.experimental.pallas import tpu as pltpu

_GRID_SIZE = 4
_ORDER = 3
_H = 0.5
_KNOTS = [_H * i - 2.5 for i in range(_GRID_SIZE + 2 * _ORDER + 1)]


def _spline_bases(y):
    b = [jnp.maximum(1.0 - 2.0 * jnp.abs(y - _KNOTS[i + 1]), 0.0)
         for i in range(len(_KNOTS) - 2)]
    for j in range(2, _ORDER + 1):
        inv = 1.0 / (j * _H)
        z = y * inv
        u = [z - _KNOTS[i] * inv for i in range(len(b))]
        b = [u[i] * b[i] + (1.0 - u[i + 1]) * b[i + 1]
             for i in range(len(b) - 1)]
    return b


def _silu_and_bases(y):
    s16 = jax.nn.silu(y).astype(jnp.bfloat16)
    bs16 = jnp.concatenate(_spline_bases(y), axis=1).astype(jnp.bfloat16)
    return s16, bs16


def _dot(a, b):
    return jnp.dot(a, b, preferred_element_type=jnp.float32)


def _make_fused_kernel(nblk, bm):
    def fused(a_ref, x16_ref, bw1_ref, sw1_ref, bw2_ref, sw2_ref,
              bwp0_ref, swp0_ref, bwp1_ref, swp1_ref, bwp2_ref, swp2_ref,
              o_ref, h16_scr, h2_scr, part_scr):
        i = pl.program_id(0)
        phase = i // nblk
        rows = pl.ds((i % nblk) * bm, bm)

        @pl.when(phase == 0)
        def _():
            y = _dot(a_ref[...], x16_ref[...])
            s16, bs16 = _silu_and_bases(y)
            h = jnp.maximum(_dot(s16, bw1_ref[...]) + _dot(bs16, sw1_ref[...]),
                            0.0)
            h16_scr[rows, :] = h.astype(jnp.bfloat16)
            part_scr[rows, :] = (_dot(s16, bwp0_ref[...])
                                 + _dot(bs16, swp0_ref[...]))

        @pl.when(phase == 1)
        def _():
            y = _dot(a_ref[...].astype(jnp.bfloat16), h16_scr[...])
            s16, bs16 = _silu_and_bases(y)
            h2 = jnp.maximum(_dot(s16, bw2_ref[...]) + _dot(bs16, sw2_ref[...]),
                             0.0)
            h2_scr[rows, :] = h2.astype(jnp.bfloat16)
            part_scr[rows, :] += (_dot(s16, bwp1_ref[...])
                                  + _dot(bs16, swp1_ref[...]))

        @pl.when(phase == 2)
        def _():
            y3 = _dot(a_ref[...].astype(jnp.bfloat16), h2_scr[...])
            s16, bs16 = _silu_and_bases(y3)
            o_ref[...] = jnp.maximum(
                part_scr[rows, :] + _dot(s16, bwp2_ref[...])
                + _dot(bs16, swp2_ref[...]), 0.0)

    return fused


def _prep_spline_w(spline_w, scaler):
    sw = spline_w * scaler[:, :, None]
    w = sw.transpose(2, 1, 0).reshape(-1, sw.shape[0])
    return w.astype(jnp.bfloat16)


def _full(shape):
    return pl.BlockSpec(shape, lambda i: (0, 0))


def kernel(x, edge_index, base_w1, spline_w1, scaler1, base_w2, spline_w2,
           scaler2, base_wo, spline_wo, scaler_o):
    n, f = x.shape
    h_dim = base_w1.shape[0]
    c_dim = base_wo.shape[0]
    bm = 400
    assert n % bm == 0
    nblk = n // bm

    x16 = x
    bw1 = base_w1.T.astype(jnp.bfloat16)
    bw2 = base_w2.T.astype(jnp.bfloat16)
    sw1 = _prep_spline_w(spline_w1, scaler1)
    sw2 = _prep_spline_w(spline_w2, scaler2)
    bwp = [base_wo[:, k * f:(k + 1) * f].T.astype(jnp.bfloat16)
           for k in range(3)]
    swp = [_prep_spline_w(spline_wo[:, k * f:(k + 1) * f, :],
                          scaler_o[:, k * f:(k + 1) * f]) for k in range(3)]

    row_blk = pl.BlockSpec((bm, n), lambda i: (i % nblk, 0))

    out = pl.pallas_call(
        _make_fused_kernel(nblk, bm),
        grid=(3 * nblk,),
        in_specs=[row_blk, _full((n, f)), _full(bw1.shape), _full(sw1.shape),
                  _full(bw2.shape), _full(sw2.shape),
                  _full(bwp[0].shape), _full(swp[0].shape),
                  _full(bwp[1].shape), _full(swp[1].shape),
                  _full(bwp[2].shape), _full(swp[2].shape)],
        out_specs=pl.BlockSpec((bm, c_dim), lambda i: (i % nblk, 0)),
        out_shape=jax.ShapeDtypeStruct((n, c_dim), jnp.float32),
        scratch_shapes=[pltpu.VMEM((n, h_dim), jnp.bfloat16),
                        pltpu.VMEM((n, h_dim), jnp.bfloat16),
                        pltpu.VMEM((n, c_dim), jnp.float32)],
    )(edge_index, x16, bw1, sw1, bw2, sw2,
      bwp[0], swp[0], bwp[1], swp[1], bwp[2], swp[2])
    return out


# speedup vs baseline: 1.0808x; 1.0020x over previous
import jax
import jax.numpy as jnp
from jax.experimental import pallas as pl
from jax.experimental.pallas import tpu as pltpu

_GRID_SIZE = 4
_ORDER = 3
_H = 0.5
_KNOTS = [_H * i - 2.5 for i in range(_GRID_SIZE + 2 * _ORDER + 1)]


def _spline_bases(y):
    b = [jnp.maximum(1.0 - 2.0 * jnp.abs(y - _KNOTS[i + 1]), 0.0)
         for i in range(len(_KNOTS) - 2)]
    for j in range(2, _ORDER + 1):
        inv = 1.0 / (j * _H)
        z = y * inv
        u = [z - _KNOTS[i] * inv for i in range(len(b))]
        b = [u[i] * b[i] + (1.0 - u[i + 1]) * b[i + 1]
             for i in range(len(b) - 1)]
    return b


def _silu_and_bases(y):
    s16 = jax.nn.silu(y).astype(jnp.bfloat16)
    bs16 = jnp.concatenate(_spline_bases(y), axis=1).astype(jnp.bfloat16)
    return s16, bs16


def _dot(a, b):
    return jnp.dot(a, b, preferred_element_type=jnp.float32)


def _make_fused_kernel(nblk, bm):
    def fused(a_ref, x16_ref, bw1_ref, sw1_ref, bw2_ref, sw2_ref,
              bwp0_ref, swp0_ref, bwp1_ref, swp1_ref, bwp2_ref, swp2_ref,
              o_ref, h16_scr, h2_scr, part_scr):
        i = pl.program_id(0)
        phase = i // nblk
        rows = pl.ds((i % nblk) * bm, bm)

        @pl.when(phase == 0)
        def _():
            y = _dot(a_ref[...], x16_ref[...])
            s16, bs16 = _silu_and_bases(y)
            h = jnp.maximum(_dot(s16, bw1_ref[...]) + _dot(bs16, sw1_ref[...]),
                            0.0)
            h16_scr[rows, :] = h
            part_scr[rows, :] = (_dot(s16, bwp0_ref[...])
                                 + _dot(bs16, swp0_ref[...]))

        @pl.when(phase == 1)
        def _():
            y = _dot(a_ref[...], h16_scr[...])
            s16, bs16 = _silu_and_bases(y)
            h2 = jnp.maximum(_dot(s16, bw2_ref[...]) + _dot(bs16, sw2_ref[...]),
                             0.0)
            h2_scr[rows, :] = h2.astype(jnp.bfloat16)
            part_scr[rows, :] += (_dot(s16, bwp1_ref[...])
                                  + _dot(bs16, swp1_ref[...]))

        @pl.when(phase == 2)
        def _():
            y3 = _dot(a_ref[...], h2_scr[...].astype(jnp.float32))
            s16, bs16 = _silu_and_bases(y3)
            o_ref[...] = jnp.maximum(
                part_scr[rows, :] + _dot(s16, bwp2_ref[...])
                + _dot(bs16, swp2_ref[...]), 0.0)

    return fused


def _prep_spline_w(spline_w, scaler):
    sw = spline_w * scaler[:, :, None]
    w = sw.transpose(2, 1, 0).reshape(-1, sw.shape[0])
    return w.astype(jnp.bfloat16)


def _full(shape):
    return pl.BlockSpec(shape, lambda i: (0, 0))


def kernel(x, edge_index, base_w1, spline_w1, scaler1, base_w2, spline_w2,
           scaler2, base_wo, spline_wo, scaler_o):
    n, f = x.shape
    h_dim = base_w1.shape[0]
    c_dim = base_wo.shape[0]
    bm = 400
    assert n % bm == 0
    nblk = n // bm

    x16 = x
    bw1 = base_w1.T.astype(jnp.bfloat16)
    bw2 = base_w2.T.astype(jnp.bfloat16)
    sw1 = _prep_spline_w(spline_w1, scaler1)
    sw2 = _prep_spline_w(spline_w2, scaler2)
    bwp = [base_wo[:, k * f:(k + 1) * f].T.astype(jnp.bfloat16)
           for k in range(3)]
    swp = [_prep_spline_w(spline_wo[:, k * f:(k + 1) * f, :],
                          scaler_o[:, k * f:(k + 1) * f]) for k in range(3)]

    row_blk = pl.BlockSpec((bm, n), lambda i: (i % nblk, 0))

    out = pl.pallas_call(
        _make_fused_kernel(nblk, bm),
        grid=(3 * nblk,),
        in_specs=[row_blk, _full((n, f)), _full(bw1.shape), _full(sw1.shape),
                  _full(bw2.shape), _full(sw2.shape),
                  _full(bwp[0].shape), _full(swp[0].shape),
                  _full(bwp[1].shape), _full(swp[1].shape),
                  _full(bwp[2].shape), _full(swp[2].shape)],
        out_specs=pl.BlockSpec((bm, c_dim), lambda i: (i % nblk, 0)),
        out_shape=jax.ShapeDtypeStruct((n, c_dim), jnp.float32),
        scratch_shapes=[pltpu.VMEM((n, h_dim), jnp.float32),
                        pltpu.VMEM((n, h_dim), jnp.bfloat16),
                        pltpu.VMEM((n, c_dim), jnp.float32)],
    )(edge_index, x16, bw1, sw1, bw2, sw2,
      bwp[0], swp[0], bwp[1], swp[1], bwp[2], swp[2])
    return out
